# Initial kernel scaffold; baseline (speedup 1.0000x reference)
import jax
import jax.numpy as jnp
from jax.experimental import pallas as pl


def kernel(x, edge_index, edge_attr, batch, params):
    raise NotImplementedError("write your pallas kernel here")


# speedup vs baseline: 2.3623x; 2.3623x over previous
import functools

import jax
import jax.numpy as jnp
from jax import lax
from jax.experimental import pallas as pl
from jax.experimental.pallas import tpu as pltpu
from jax.experimental.pallas import tpu_sc as plsc

N = 10000
E = 320000
D_IN = 128
D_EDGE = 16
H = 64
DEC = 64
OUT = 1
G = 256
NUM_LAYERS = 3

NC = 2
NS = 16
NW = NC * NS
EPW = E // NW
CH = 80
NCH = EPW // CH
NPS = 624
TAIL = N - NS * NPS

f32 = jnp.float32


@functools.lru_cache(maxsize=1)
def _sc_kernels():
    mesh = plsc.VectorSubcoreMesh(core_axis_name="c", subcore_axis_name="s")

    @functools.partial(
        pl.kernel,
        out_type=(
            jax.ShapeDtypeStruct((E, H), f32),
            jax.ShapeDtypeStruct((E, H), f32),
        ),
        mesh=mesh,
        scratch_types=[
            pltpu.VMEM((NCH, CH), jnp.int32),
            pltpu.VMEM((NCH, CH), jnp.int32),
            pltpu.VMEM((CH, H), f32),
            pltpu.VMEM((CH, H), f32),
            pltpu.SemaphoreType.DMA,
        ],
        compiler_params=pltpu.CompilerParams(use_tc_tiling_on_sc=False),
    )
    def sc_gather(p_hbm, q_hbm, dst_hbm, src_hbm, outd_hbm, outs_hbm,
                  dst_v, src_v, bufd, bufs, sem):
        wid = lax.axis_index("s") * NC + lax.axis_index("c")
        pltpu.sync_copy(dst_hbm.at[wid], dst_v)
        pltpu.sync_copy(src_hbm.at[wid], src_v)

        def body(j, carry):
            cp1 = pltpu.async_copy(p_hbm.at[dst_v.at[j]], bufd, sem)
            cp2 = pltpu.async_copy(q_hbm.at[src_v.at[j]], bufs, sem)
            cp1.wait()
            cp2.wait()
            base = wid * EPW + j * CH
            pltpu.sync_copy(bufd, outd_hbm.at[pl.ds(base, CH)])
            pltpu.sync_copy(bufs, outs_hbm.at[pl.ds(base, CH)])
            return carry

        lax.fori_loop(0, NCH, body, 0)

    @functools.partial(
        pl.kernel,
        out_type=jax.ShapeDtypeStruct((NC, N, H), f32),
        mesh=mesh,
        scratch_types=[
            pltpu.VMEM((NCH, CH), jnp.int32),
            pltpu.VMEM((CH, H), f32),
            pltpu.VMEM_SHARED((N, H), f32),
            pltpu.SemaphoreType.DMA,
        ],
        compiler_params=pltpu.CompilerParams(use_tc_tiling_on_sc=False),
    )
    def sc_scatter(m_hbm, dst_hbm, zeros_hbm, out_hbm, dst_v, buf, acc, sem):
        cid = lax.axis_index("c")
        sid = lax.axis_index("s")
        wid = sid * NC + cid
        r0 = sid * NPS
        pltpu.sync_copy(zeros_hbm.at[pl.ds(r0, NPS)], acc.at[pl.ds(r0, NPS)])

        @pl.when(sid == NS - 1)
        def _():
            pltpu.sync_copy(zeros_hbm.at[pl.ds(NS * NPS, TAIL)],
                            acc.at[pl.ds(NS * NPS, TAIL)])

        plsc.subcore_barrier()

        pltpu.sync_copy(dst_hbm.at[wid], dst_v)

        def body(j, carry):
            base = wid * EPW + j * CH
            pltpu.sync_copy(m_hbm.at[pl.ds(base, CH)], buf)
            pltpu.sync_copy(buf, acc.at[dst_v.at[j]], add=True)
            return carry

        lax.fori_loop(0, NCH, body, 0)
        plsc.subcore_barrier()
        pltpu.sync_copy(acc.at[pl.ds(r0, NPS)], out_hbm.at[cid, pl.ds(r0, NPS)])

        @pl.when(sid == NS - 1)
        def _():
            pltpu.sync_copy(acc.at[pl.ds(NS * NPS, TAIL)],
                            out_hbm.at[cid, pl.ds(NS * NPS, TAIL)])

    @functools.partial(
        pl.kernel,
        out_type=jax.ShapeDtypeStruct((NC, N, 16), f32),
        mesh=mesh,
        scratch_types=[
            pltpu.VMEM((NCH, CH), jnp.int32),
            pltpu.VMEM((CH, 16), f32),
            pltpu.VMEM_SHARED((N, 16), f32),
            pltpu.SemaphoreType.DMA,
        ],
        compiler_params=pltpu.CompilerParams(use_tc_tiling_on_sc=False),
    )
    def sc_counts(dst_hbm, ones_hbm, zeros_hbm, out_hbm, dst_v, buf, acc, sem):
        cid = lax.axis_index("c")
        sid = lax.axis_index("s")
        wid = sid * NC + cid
        r0 = sid * NPS
        pltpu.sync_copy(zeros_hbm.at[pl.ds(r0, NPS)], acc.at[pl.ds(r0, NPS)])

        @pl.when(sid == NS - 1)
        def _():
            pltpu.sync_copy(zeros_hbm.at[pl.ds(NS * NPS, TAIL)],
                            acc.at[pl.ds(NS * NPS, TAIL)])

        plsc.subcore_barrier()

        pltpu.sync_copy(dst_hbm.at[wid], dst_v)
        pltpu.sync_copy(ones_hbm, buf)

        def body(j, carry):
            pltpu.sync_copy(buf, acc.at[dst_v.at[j]], add=True)
            return carry

        lax.fori_loop(0, NCH, body, 0)
        plsc.subcore_barrier()
        pltpu.sync_copy(acc.at[pl.ds(r0, NPS)], out_hbm.at[cid, pl.ds(r0, NPS)])

        @pl.when(sid == NS - 1)
        def _():
            pltpu.sync_copy(acc.at[pl.ds(NS * NPS, TAIL)],
                            out_hbm.at[cid, pl.ds(NS * NPS, TAIL)])

    return sc_gather, sc_scatter, sc_counts


def _sc_gather(p, q, dst2d, src2d):
    return _sc_kernels()[0](p, q, dst2d, src2d)


def _sc_scatter(m, dst2d, zeros_h):
    return _sc_kernels()[1](m, dst2d, zeros_h)


def _sc_counts(dst2d, ones_ch, zeros_16):
    return _sc_kernels()[2](dst2d, ones_ch, zeros_16)


def _prep_body(x_ref, wa_ref, wb_ref, p_ref, q_ref):
    x = x_ref[...]
    p_ref[...] = jnp.dot(x, wa_ref[...], preferred_element_type=f32)
    q_ref[...] = jnp.dot(x, wb_ref[...], preferred_element_type=f32)


def _tc_prep(x, wa, wb):
    bn = 2000
    cin = x.shape[1]
    return pl.pallas_call(
        _prep_body,
        grid=(N // bn,),
        in_specs=[
            pl.BlockSpec((bn, cin), lambda i: (i, 0)),
            pl.BlockSpec((cin, H), lambda i: (0, 0)),
            pl.BlockSpec((cin, H), lambda i: (0, 0)),
        ],
        out_specs=[
            pl.BlockSpec((bn, H), lambda i: (i, 0)),
            pl.BlockSpec((bn, H), lambda i: (i, 0)),
        ],
        out_shape=[
            jax.ShapeDtypeStruct((N, H), f32),
            jax.ShapeDtypeStruct((N, H), f32),
        ],
    )(x, wa, wb)


def _edge_body(gd_ref, gs_ref, ea_ref, wc_ref, b0_ref, w1_ref, b1_ref,
               wg_ref, bg_ref, m_ref):
    ea = ea_ref[...]
    pre = (gd_ref[...] + gs_ref[...]
           + jnp.dot(ea, wc_ref[...], preferred_element_type=f32)
           + b0_ref[...])
    h = jnp.maximum(pre, 0.0)
    msg = jnp.dot(h, w1_ref[...], preferred_element_type=f32) + b1_ref[...]
    gate = jax.nn.sigmoid(
        jnp.dot(ea, wg_ref[...], preferred_element_type=f32) + bg_ref[...])
    m_ref[...] = msg * gate


def _tc_edge(gd, gs, ea, wc, b0, w1, b1, wg, bg):
    be = 2000
    return pl.pallas_call(
        _edge_body,
        grid=(E // be,),
        in_specs=[
            pl.BlockSpec((be, H), lambda i: (i, 0)),
            pl.BlockSpec((be, H), lambda i: (i, 0)),
            pl.BlockSpec((be, D_EDGE), lambda i: (i, 0)),
            pl.BlockSpec((D_EDGE, H), lambda i: (0, 0)),
            pl.BlockSpec((1, H), lambda i: (0, 0)),
            pl.BlockSpec((H, H), lambda i: (0, 0)),
            pl.BlockSpec((1, H), lambda i: (0, 0)),
            pl.BlockSpec((D_EDGE, H), lambda i: (0, 0)),
            pl.BlockSpec((1, H), lambda i: (0, 0)),
        ],
        out_specs=pl.BlockSpec((be, H), lambda i: (i, 0)),
        out_shape=jax.ShapeDtypeStruct((E, H), f32),
    )(gd, gs, ea, wc, b0, w1, b1, wg, bg)


def _make_node_body(has_res, has_next):
    def body(*refs):
        it = iter(refs)
        x_ref = next(it)
        s_ref = next(it)
        cnt_ref = next(it)
        wnx_ref = next(it)
        wna_ref = next(it)
        bn0_ref = next(it)
        wn1_ref = next(it)
        bn1_ref = next(it)
        if has_res:
            rw_ref = next(it)
            rb_ref = next(it)
        g_ref = next(it)
        b_ref = next(it)
        if has_next:
            wa_ref = next(it)
            wb_ref = next(it)
        h_ref = next(it)
        if has_next:
            p_ref = next(it)
            q_ref = next(it)

        x = x_ref[...]
        sums = s_ref[0] + s_ref[1]
        cnt = cnt_ref[0, :, 0:1] + cnt_ref[1, :, 0:1]
        aggr = sums / jnp.maximum(cnt, 1.0)
        u = jnp.maximum(
            jnp.dot(x, wnx_ref[...], preferred_element_type=f32)
            + jnp.dot(aggr, wna_ref[...], preferred_element_type=f32)
            + bn0_ref[...], 0.0)
        out = jnp.dot(u, wn1_ref[...], preferred_element_type=f32) + bn1_ref[...]
        if has_res:
            res = jnp.dot(x, rw_ref[...], preferred_element_type=f32) + rb_ref[...]
        else:
            res = x
        z = out + res
        mu = jnp.mean(z, axis=-1, keepdims=True)
        var = jnp.mean((z - mu) * (z - mu), axis=-1, keepdims=True)
        zn = (z - mu) * lax.rsqrt(var + 1e-5) * g_ref[...] + b_ref[...]
        h = jnp.maximum(zn, 0.0)
        h_ref[...] = h
        if has_next:
            p_ref[...] = jnp.dot(h, wa_ref[...], preferred_element_type=f32)
            q_ref[...] = jnp.dot(h, wb_ref[...], preferred_element_type=f32)
    return body


def _tc_node(x, s01, cnts, p, has_res, nxt):
    bn = 2000
    cin = x.shape[1]
    s = s01
    in_specs = [
        pl.BlockSpec((bn, cin), lambda i: (i, 0)),
        pl.BlockSpec((NC, bn, H), lambda i: (0, i, 0)),
        pl.BlockSpec((NC, bn, 16), lambda i: (0, i, 0)),
        pl.BlockSpec((cin, H), lambda i: (0, 0)),
        pl.BlockSpec((H, H), lambda i: (0, 0)),
        pl.BlockSpec((1, H), lambda i: (0, 0)),
        pl.BlockSpec((H, H), lambda i: (0, 0)),
        pl.BlockSpec((1, H), lambda i: (0, 0)),
    ]
    wnx = p['node_w0'][:cin]
    wna = p['node_w0'][cin:]
    args = [x, s, cnts, wnx, wna, p['node_b0'].reshape(1, H),
            p['node_w1'], p['node_b1'].reshape(1, H)]
    if has_res:
        in_specs += [
            pl.BlockSpec((cin, H), lambda i: (0, 0)),
            pl.BlockSpec((1, H), lambda i: (0, 0)),
        ]
        args += [p['res_w'], p['res_b'].reshape(1, H)]
    in_specs += [
        pl.BlockSpec((1, H), lambda i: (0, 0)),
        pl.BlockSpec((1, H), lambda i: (0, 0)),
    ]
    args += [p['ln_g'].reshape(1, H), p['ln_b'].reshape(1, H)]
    out_specs = [pl.BlockSpec((bn, H), lambda i: (i, 0))]
    out_shape = [jax.ShapeDtypeStruct((N, H), f32)]
    if nxt is not None:
        wa_n, wb_n = nxt
        in_specs += [
            pl.BlockSpec((H, H), lambda i: (0, 0)),
            pl.BlockSpec((H, H), lambda i: (0, 0)),
        ]
        args += [wa_n, wb_n]
        out_specs += [
            pl.BlockSpec((bn, H), lambda i: (i, 0)),
            pl.BlockSpec((bn, H), lambda i: (i, 0)),
        ]
        out_shape += [
            jax.ShapeDtypeStruct((N, H), f32),
            jax.ShapeDtypeStruct((N, H), f32),
        ]
    return pl.pallas_call(
        _make_node_body(has_res, nxt is not None),
        grid=(N // bn,),
        in_specs=in_specs,
        out_specs=out_specs,
        out_shape=out_shape,
    )(*args)


def _readout_body(h_ref, b_ref, w0_ref, b0_ref, w1_ref, b1_ref, o_ref):
    h = h_ref[...]
    ids = b_ref[...]
    onehot = (ids == lax.broadcasted_iota(jnp.int32, (1, G), 1)).astype(f32)
    sums = lax.dot_general(onehot, h, (((0,), (0,)), ((), ())),
                           preferred_element_type=f32,
                           precision=lax.Precision.HIGHEST)
    cnt = jnp.sum(onehot, axis=0, keepdims=True)
    hg = sums / jnp.maximum(cnt.T, 1.0)
    o = jnp.maximum(
        jnp.dot(hg, w0_ref[...], preferred_element_type=f32) + b0_ref[...],
        0.0)
    o_ref[...] = jnp.dot(o, w1_ref[...], preferred_element_type=f32) + b1_ref[...]


def _tc_readout(h, batch2d, r):
    return pl.pallas_call(
        _readout_body,
        grid=(1,),
        in_specs=[
            pl.BlockSpec((N, H), lambda i: (0, 0)),
            pl.BlockSpec((N, 1), lambda i: (0, 0)),
            pl.BlockSpec((H, DEC), lambda i: (0, 0)),
            pl.BlockSpec((1, DEC), lambda i: (0, 0)),
            pl.BlockSpec((DEC, OUT), lambda i: (0, 0)),
            pl.BlockSpec((1, OUT), lambda i: (0, 0)),
        ],
        out_specs=pl.BlockSpec((G, OUT), lambda i: (0, 0)),
        out_shape=jax.ShapeDtypeStruct((G, OUT), f32),
    )(h, batch2d, r['w0'], r['b0'].reshape(1, DEC), r['w1'],
      r['b1'].reshape(1, OUT))


def kernel(x, edge_index, edge_attr, batch, params):
    src = edge_index[0]
    dst = edge_index[1]
    dst2d = dst.reshape(NW, NCH, CH)
    src2d = src.reshape(NW, NCH, CH)

    zeros_h = jnp.zeros((N, H), f32)
    zeros_16 = jnp.zeros((N, 16), f32)
    ones_ch = jnp.ones((CH, 16), f32)

    cnts = _sc_counts(dst2d, ones_ch, zeros_16)

    h = x
    p0 = params['layer0']
    cin0 = D_IN
    wa = p0['msg_w0'][:cin0]
    wb = p0['msg_w0'][cin0:2 * cin0]
    P, Q = _tc_prep(x, wa, wb)

    for l in range(NUM_LAYERS):
        p = params['layer%d' % l]
        cin = D_IN if l == 0 else H
        wc = p['msg_w0'][2 * cin:]
        gd, gs = _sc_gather(P, Q, dst2d, src2d)
        m = _tc_edge(gd, gs, edge_attr, wc, p['msg_b0'].reshape(1, H),
                     p['msg_w1'], p['msg_b1'].reshape(1, H),
                     p['gate_w'], p['gate_b'].reshape(1, H))
        sums = _sc_scatter(m, dst2d, zeros_h)
        if l + 1 < NUM_LAYERS:
            pn = params['layer%d' % (l + 1)]
            nxt = (pn['msg_w0'][:H], pn['msg_w0'][H:2 * H])
            h, P, Q = _tc_node(h, sums, cnts, p, l == 0, nxt)
        else:
            (h,) = _tc_node(h, sums, cnts, p, l == 0, None)

    return _tc_readout(h, batch.reshape(N, 1), params['readout'])


# speedup vs baseline: 3.9925x; 1.6901x over previous
import functools

import jax
import jax.numpy as jnp
from jax import lax
from jax.experimental import pallas as pl
from jax.experimental.pallas import tpu as pltpu
from jax.experimental.pallas import tpu_sc as plsc

N = 10000
E = 320000
D_IN = 128
D_EDGE = 16
H = 64
DEC = 64
OUT = 1
G = 256
NUM_LAYERS = 3

NC = 2
NS = 16
NW = NC * NS
EPW = E // NW
CH = 80
NCH = EPW // CH
NPS = 624
TAIL = N - NS * NPS

f32 = jnp.float32


@functools.lru_cache(maxsize=1)
def _sc_kernels():
    mesh = plsc.VectorSubcoreMesh(core_axis_name="c", subcore_axis_name="s")

    @functools.partial(
        pl.kernel,
        out_type=(
            jax.ShapeDtypeStruct((E, H), f32),
            jax.ShapeDtypeStruct((E, H), f32),
        ),
        mesh=mesh,
        scratch_types=[
            pltpu.VMEM((NCH, CH), jnp.int32),
            pltpu.VMEM((NCH, CH), jnp.int32),
            pltpu.VMEM((CH, H), f32),
            pltpu.VMEM((CH, H), f32),
            pltpu.SemaphoreType.DMA,
        ],
        compiler_params=pltpu.CompilerParams(use_tc_tiling_on_sc=False),
    )
    def sc_gather(p_hbm, q_hbm, dst_hbm, src_hbm, outd_hbm, outs_hbm,
                  dst_v, src_v, bufd, bufs, sem):
        wid = lax.axis_index("s") * NC + lax.axis_index("c")
        pltpu.sync_copy(dst_hbm.at[wid], dst_v)
        pltpu.sync_copy(src_hbm.at[wid], src_v)

        def body(j, carry):
            cp1 = pltpu.async_copy(p_hbm.at[dst_v.at[j]], bufd, sem)
            cp2 = pltpu.async_copy(q_hbm.at[src_v.at[j]], bufs, sem)
            cp1.wait()
            cp2.wait()
            base = wid * EPW + j * CH
            pltpu.sync_copy(bufd, outd_hbm.at[pl.ds(base, CH)])
            pltpu.sync_copy(bufs, outs_hbm.at[pl.ds(base, CH)])
            return carry

        lax.fori_loop(0, NCH, body, 0)

    @functools.partial(
        pl.kernel,
        out_type=jax.ShapeDtypeStruct((NC, N, H), f32),
        mesh=mesh,
        scratch_types=[
            pltpu.VMEM((NCH, CH), jnp.int32),
            pltpu.VMEM((CH, H), f32),
            pltpu.VMEM_SHARED((N, H), f32),
            pltpu.SemaphoreType.DMA,
        ],
        compiler_params=pltpu.CompilerParams(use_tc_tiling_on_sc=False),
    )
    def sc_scatter(m_hbm, dst_hbm, zeros_hbm, out_hbm, dst_v, buf, acc, sem):
        cid = lax.axis_index("c")
        sid = lax.axis_index("s")
        wid = sid * NC + cid
        r0 = sid * NPS
        pltpu.sync_copy(zeros_hbm.at[pl.ds(r0, NPS)], acc.at[pl.ds(r0, NPS)])

        @pl.when(sid == NS - 1)
        def _():
            pltpu.sync_copy(zeros_hbm.at[pl.ds(NS * NPS, TAIL)],
                            acc.at[pl.ds(NS * NPS, TAIL)])

        plsc.subcore_barrier()

        pltpu.sync_copy(dst_hbm.at[wid], dst_v)

        def body(j, carry):
            base = wid * EPW + j * CH
            pltpu.sync_copy(m_hbm.at[pl.ds(base, CH)], buf)
            pltpu.sync_copy(buf, acc.at[dst_v.at[j]], add=True)
            return carry

        lax.fori_loop(0, NCH, body, 0)
        plsc.subcore_barrier()
        pltpu.sync_copy(acc.at[pl.ds(r0, NPS)], out_hbm.at[cid, pl.ds(r0, NPS)])

        @pl.when(sid == NS - 1)
        def _():
            pltpu.sync_copy(acc.at[pl.ds(NS * NPS, TAIL)],
                            out_hbm.at[cid, pl.ds(NS * NPS, TAIL)])

    @functools.partial(
        pl.kernel,
        out_type=jax.ShapeDtypeStruct((NC, N, 16), f32),
        mesh=mesh,
        scratch_types=[
            pltpu.VMEM((NCH, CH), jnp.int32),
            pltpu.VMEM((CH, 16), f32),
            pltpu.VMEM_SHARED((N, 16), f32),
            pltpu.SemaphoreType.DMA,
        ],
        compiler_params=pltpu.CompilerParams(use_tc_tiling_on_sc=False),
    )
    def sc_counts(dst_hbm, ones_hbm, zeros_hbm, out_hbm, dst_v, buf, acc, sem):
        cid = lax.axis_index("c")
        sid = lax.axis_index("s")
        wid = sid * NC + cid
        r0 = sid * NPS
        pltpu.sync_copy(zeros_hbm.at[pl.ds(r0, NPS)], acc.at[pl.ds(r0, NPS)])

        @pl.when(sid == NS - 1)
        def _():
            pltpu.sync_copy(zeros_hbm.at[pl.ds(NS * NPS, TAIL)],
                            acc.at[pl.ds(NS * NPS, TAIL)])

        plsc.subcore_barrier()

        pltpu.sync_copy(dst_hbm.at[wid], dst_v)
        pltpu.sync_copy(ones_hbm, buf)

        def body(j, carry):
            pltpu.sync_copy(buf, acc.at[dst_v.at[j]], add=True)
            return carry

        lax.fori_loop(0, NCH, body, 0)
        plsc.subcore_barrier()
        pltpu.sync_copy(acc.at[pl.ds(r0, NPS)], out_hbm.at[cid, pl.ds(r0, NPS)])

        @pl.when(sid == NS - 1)
        def _():
            pltpu.sync_copy(acc.at[pl.ds(NS * NPS, TAIL)],
                            out_hbm.at[cid, pl.ds(NS * NPS, TAIL)])

    return sc_gather, sc_scatter, sc_counts


def _sc_gather(p, q, dst2d, src2d):
    return _sc_kernels()[0](p, q, dst2d, src2d)


def _sc_scatter(m, dst2d, zeros_h):
    return _sc_kernels()[1](m, dst2d, zeros_h)


def _sc_counts(dst2d, ones_ch, zeros_16):
    return _sc_kernels()[2](dst2d, ones_ch, zeros_16)


def _prep_body(x_ref, wa_ref, wb_ref, p_ref, q_ref):
    x = x_ref[...]
    p_ref[...] = jnp.dot(x, wa_ref[...], preferred_element_type=f32)
    q_ref[...] = jnp.dot(x, wb_ref[...], preferred_element_type=f32)


def _tc_prep(x, wa, wb):
    bn = 2000
    cin = x.shape[1]
    return pl.pallas_call(
        _prep_body,
        grid=(N // bn,),
        in_specs=[
            pl.BlockSpec((bn, cin), lambda i: (i, 0)),
            pl.BlockSpec((cin, H), lambda i: (0, 0)),
            pl.BlockSpec((cin, H), lambda i: (0, 0)),
        ],
        out_specs=[
            pl.BlockSpec((bn, H), lambda i: (i, 0)),
            pl.BlockSpec((bn, H), lambda i: (i, 0)),
        ],
        out_shape=[
            jax.ShapeDtypeStruct((N, H), f32),
            jax.ShapeDtypeStruct((N, H), f32),
        ],
    )(x, wa, wb)


def _edge_body(gd_ref, gs_ref, ea_ref, wc_ref, b0_ref, w1_ref, b1_ref,
               wg_ref, bg_ref, m_ref):
    ea = ea_ref[...]
    pre = (gd_ref[...] + gs_ref[...]
           + jnp.dot(ea, wc_ref[...], preferred_element_type=f32)
           + b0_ref[...])
    h = jnp.maximum(pre, 0.0)
    msg = jnp.dot(h, w1_ref[...], preferred_element_type=f32) + b1_ref[...]
    gate = jax.nn.sigmoid(
        jnp.dot(ea, wg_ref[...], preferred_element_type=f32) + bg_ref[...])
    m_ref[...] = msg * gate


def _tc_edge(gd, gs, ea2, wc2, b02, w12, b12, wg2, bg2):
    be = 1000
    e2 = E // 2
    return pl.pallas_call(
        _edge_body,
        grid=(e2 // be,),
        in_specs=[
            pl.BlockSpec((be, 128), lambda i: (i, 0)),
            pl.BlockSpec((be, 128), lambda i: (i, 0)),
            pl.BlockSpec((be, 2 * D_EDGE), lambda i: (i, 0)),
            pl.BlockSpec((2 * D_EDGE, 128), lambda i: (0, 0)),
            pl.BlockSpec((1, 128), lambda i: (0, 0)),
            pl.BlockSpec((128, 128), lambda i: (0, 0)),
            pl.BlockSpec((1, 128), lambda i: (0, 0)),
            pl.BlockSpec((2 * D_EDGE, 128), lambda i: (0, 0)),
            pl.BlockSpec((1, 128), lambda i: (0, 0)),
        ],
        out_specs=pl.BlockSpec((be, 128), lambda i: (i, 0)),
        out_shape=jax.ShapeDtypeStruct((e2, 128), f32),
    )(gd, gs, ea2, wc2, b02, w12, b12, wg2, bg2)


def _make_node_body(has_res, has_next):
    def body(*refs):
        it = iter(refs)
        x_ref = next(it)
        s_ref = next(it)
        cnt_ref = next(it)
        wnx_ref = next(it)
        wna_ref = next(it)
        bn0_ref = next(it)
        wn1_ref = next(it)
        bn1_ref = next(it)
        if has_res:
            rw_ref = next(it)
            rb_ref = next(it)
        g_ref = next(it)
        b_ref = next(it)
        if has_next:
            wa_ref = next(it)
            wb_ref = next(it)
        h_ref = next(it)
        if has_next:
            p_ref = next(it)
            q_ref = next(it)

        x = x_ref[...]
        sums = s_ref[0] + s_ref[1]
        cnt = cnt_ref[0, :, 0:1] + cnt_ref[1, :, 0:1]
        aggr = sums / jnp.maximum(cnt, 1.0)
        u = jnp.maximum(
            jnp.dot(x, wnx_ref[...], preferred_element_type=f32)
            + jnp.dot(aggr, wna_ref[...], preferred_element_type=f32)
            + bn0_ref[...], 0.0)
        out = jnp.dot(u, wn1_ref[...], preferred_element_type=f32) + bn1_ref[...]
        if has_res:
            res = jnp.dot(x, rw_ref[...], preferred_element_type=f32) + rb_ref[...]
        else:
            res = x
        z = out + res
        mu = jnp.mean(z, axis=-1, keepdims=True)
        var = jnp.mean((z - mu) * (z - mu), axis=-1, keepdims=True)
        zn = (z - mu) * lax.rsqrt(var + 1e-5) * g_ref[...] + b_ref[...]
        h = jnp.maximum(zn, 0.0)
        h_ref[...] = h
        if has_next:
            p_ref[...] = jnp.dot(h, wa_ref[...], preferred_element_type=f32)
            q_ref[...] = jnp.dot(h, wb_ref[...], preferred_element_type=f32)
    return body


def _tc_node(x, s01, cnts, p, has_res, nxt):
    bn = 2000
    cin = x.shape[1]
    s = s01
    in_specs = [
        pl.BlockSpec((bn, cin), lambda i: (i, 0)),
        pl.BlockSpec((NC, bn, H), lambda i: (0, i, 0)),
        pl.BlockSpec((NC, bn, 16), lambda i: (0, i, 0)),
        pl.BlockSpec((cin, H), lambda i: (0, 0)),
        pl.BlockSpec((H, H), lambda i: (0, 0)),
        pl.BlockSpec((1, H), lambda i: (0, 0)),
        pl.BlockSpec((H, H), lambda i: (0, 0)),
        pl.BlockSpec((1, H), lambda i: (0, 0)),
    ]
    wnx = p['node_w0'][:cin]
    wna = p['node_w0'][cin:]
    args = [x, s, cnts, wnx, wna, p['node_b0'].reshape(1, H),
            p['node_w1'], p['node_b1'].reshape(1, H)]
    if has_res:
        in_specs += [
            pl.BlockSpec((cin, H), lambda i: (0, 0)),
            pl.BlockSpec((1, H), lambda i: (0, 0)),
        ]
        args += [p['res_w'], p['res_b'].reshape(1, H)]
    in_specs += [
        pl.BlockSpec((1, H), lambda i: (0, 0)),
        pl.BlockSpec((1, H), lambda i: (0, 0)),
    ]
    args += [p['ln_g'].reshape(1, H), p['ln_b'].reshape(1, H)]
    out_specs = [pl.BlockSpec((bn, H), lambda i: (i, 0))]
    out_shape = [jax.ShapeDtypeStruct((N, H), f32)]
    if nxt is not None:
        wa_n, wb_n = nxt
        in_specs += [
            pl.BlockSpec((H, H), lambda i: (0, 0)),
            pl.BlockSpec((H, H), lambda i: (0, 0)),
        ]
        args += [wa_n, wb_n]
        out_specs += [
            pl.BlockSpec((bn, H), lambda i: (i, 0)),
            pl.BlockSpec((bn, H), lambda i: (i, 0)),
        ]
        out_shape += [
            jax.ShapeDtypeStruct((N, H), f32),
            jax.ShapeDtypeStruct((N, H), f32),
        ]
    return pl.pallas_call(
        _make_node_body(has_res, nxt is not None),
        grid=(N // bn,),
        in_specs=in_specs,
        out_specs=out_specs,
        out_shape=out_shape,
    )(*args)


def _readout_body(h_ref, b_ref, w0_ref, b0_ref, w1_ref, b1_ref, o_ref):
    h = h_ref[...]
    ids = b_ref[...]
    onehot = (ids == lax.broadcasted_iota(jnp.int32, (1, G), 1)).astype(f32)
    sums = lax.dot_general(onehot, h, (((0,), (0,)), ((), ())),
                           preferred_element_type=f32,
                           precision=lax.Precision.HIGHEST)
    cnt = jnp.sum(onehot, axis=0, keepdims=True)
    hg = sums / jnp.maximum(cnt.T, 1.0)
    o = jnp.maximum(
        jnp.dot(hg, w0_ref[...], preferred_element_type=f32) + b0_ref[...],
        0.0)
    o_ref[...] = jnp.dot(o, w1_ref[...], preferred_element_type=f32) + b1_ref[...]


def _tc_readout(h, batch2d, r):
    return pl.pallas_call(
        _readout_body,
        grid=(1,),
        in_specs=[
            pl.BlockSpec((N, H), lambda i: (0, 0)),
            pl.BlockSpec((N, 1), lambda i: (0, 0)),
            pl.BlockSpec((H, DEC), lambda i: (0, 0)),
            pl.BlockSpec((1, DEC), lambda i: (0, 0)),
            pl.BlockSpec((DEC, OUT), lambda i: (0, 0)),
            pl.BlockSpec((1, OUT), lambda i: (0, 0)),
        ],
        out_specs=pl.BlockSpec((G, OUT), lambda i: (0, 0)),
        out_shape=jax.ShapeDtypeStruct((G, OUT), f32),
    )(h, batch2d, r['w0'], r['b0'].reshape(1, DEC), r['w1'],
      r['b1'].reshape(1, OUT))


def _blockdiag(w):
    z = jnp.zeros_like(w)
    top = jnp.concatenate([w, z], axis=1)
    bot = jnp.concatenate([z, w], axis=1)
    return jnp.concatenate([top, bot], axis=0)


def _dup(b):
    return jnp.concatenate([b, b]).reshape(1, -1)


def kernel(x, edge_index, edge_attr, batch, params):
    src = edge_index[0]
    dst = edge_index[1]
    dst2d = dst.reshape(NW, NCH, CH)
    src2d = src.reshape(NW, NCH, CH)

    ea2 = edge_attr.reshape(E // 2, 2 * D_EDGE)
    zeros_h = jnp.zeros((N, H), f32)
    zeros_16 = jnp.zeros((N, 16), f32)
    ones_ch = jnp.ones((CH, 16), f32)

    cnts = _sc_counts(dst2d, ones_ch, zeros_16)

    h = x
    p0 = params['layer0']
    cin0 = D_IN
    wa = p0['msg_w0'][:cin0]
    wb = p0['msg_w0'][cin0:2 * cin0]
    P, Q = _tc_prep(x, wa, wb)

    for l in range(NUM_LAYERS):
        p = params['layer%d' % l]
        cin = D_IN if l == 0 else H
        wc = p['msg_w0'][2 * cin:]
        gd, gs = _sc_gather(P, Q, dst2d, src2d)
        m = _tc_edge(gd.reshape(E // 2, 128), gs.reshape(E // 2, 128), ea2,
                     _blockdiag(wc), _dup(p['msg_b0']),
                     _blockdiag(p['msg_w1']), _dup(p['msg_b1']),
                     _blockdiag(p['gate_w']), _dup(p['gate_b']))
        sums = _sc_scatter(m.reshape(E, H), dst2d, zeros_h)
        if l + 1 < NUM_LAYERS:
            pn = params['layer%d' % (l + 1)]
            nxt = (pn['msg_w0'][:H], pn['msg_w0'][H:2 * H])
            h, P, Q = _tc_node(h, sums, cnts, p, l == 0, nxt)
        else:
            (h,) = _tc_node(h, sums, cnts, p, l == 0, None)

    return _tc_readout(h, batch.reshape(N, 1), params['readout'])


# speedup vs baseline: 5.1812x; 1.2977x over previous
import functools

import jax
import jax.numpy as jnp
from jax import lax
from jax.experimental import pallas as pl
from jax.experimental.pallas import tpu as pltpu
from jax.experimental.pallas import tpu_sc as plsc

N = 10000
E = 320000
D_IN = 128
D_EDGE = 16
H = 64
DEC = 64
OUT = 1
G = 256
NUM_LAYERS = 3

NC = 2
NS = 16
NW = NC * NS
EPW = E // NW
CH = 500
NCH = EPW // CH
CHS = 1000
NCHS = EPW // CHS
NPS = 624
TAIL = N - NS * NPS

f32 = jnp.float32


@functools.lru_cache(maxsize=1)
def _sc_kernels():
    mesh = plsc.VectorSubcoreMesh(core_axis_name="c", subcore_axis_name="s")

    @functools.partial(
        pl.kernel,
        out_type=(
            jax.ShapeDtypeStruct((E, H), f32),
            jax.ShapeDtypeStruct((E, H), f32),
        ),
        mesh=mesh,
        scratch_types=[
            pltpu.VMEM((NCH, CH), jnp.int32),
            pltpu.VMEM((NCH, CH), jnp.int32),
            pltpu.VMEM((CH, H), f32),
            pltpu.VMEM((CH, H), f32),
            pltpu.SemaphoreType.DMA,
        ],
        compiler_params=pltpu.CompilerParams(use_tc_tiling_on_sc=False),
    )
    def sc_gather(p_hbm, q_hbm, dst_hbm, src_hbm, outd_hbm, outs_hbm,
                  dst_v, src_v, bufd, bufs, sem):
        wid = lax.axis_index("s") * NC + lax.axis_index("c")
        pltpu.sync_copy(dst_hbm.at[wid], dst_v)
        pltpu.sync_copy(src_hbm.at[wid], src_v)

        def body(j, carry):
            cp1 = pltpu.async_copy(p_hbm.at[dst_v.at[j]], bufd, sem)
            cp2 = pltpu.async_copy(q_hbm.at[src_v.at[j]], bufs, sem)
            cp1.wait()
            cp2.wait()
            base = wid * EPW + j * CH
            pltpu.sync_copy(bufd, outd_hbm.at[pl.ds(base, CH)])
            pltpu.sync_copy(bufs, outs_hbm.at[pl.ds(base, CH)])
            return carry

        lax.fori_loop(0, NCH, body, 0)

    @functools.partial(
        pl.kernel,
        out_type=jax.ShapeDtypeStruct((NC, N, H), f32),
        mesh=mesh,
        scratch_types=[
            pltpu.VMEM((NCHS, CHS), jnp.int32),
            pltpu.VMEM((CHS, H), f32),
            pltpu.VMEM_SHARED((N, H), f32),
            pltpu.SemaphoreType.DMA,
        ],
        compiler_params=pltpu.CompilerParams(use_tc_tiling_on_sc=False),
    )
    def sc_scatter(m_hbm, dst_hbm, zeros_hbm, out_hbm, dst_v, buf, acc, sem):
        cid = lax.axis_index("c")
        sid = lax.axis_index("s")
        wid = sid * NC + cid
        r0 = sid * NPS
        pltpu.sync_copy(zeros_hbm.at[pl.ds(r0, NPS)], acc.at[pl.ds(r0, NPS)])

        @pl.when(sid == NS - 1)
        def _():
            pltpu.sync_copy(zeros_hbm.at[pl.ds(NS * NPS, TAIL)],
                            acc.at[pl.ds(NS * NPS, TAIL)])

        plsc.subcore_barrier()

        pltpu.sync_copy(dst_hbm.at[wid], dst_v)

        def body(j, carry):
            base = wid * EPW + j * CHS
            pltpu.sync_copy(m_hbm.at[pl.ds(base, CHS)], buf)
            pltpu.sync_copy(buf, acc.at[dst_v.at[j]], add=True)
            return carry

        lax.fori_loop(0, NCHS, body, 0)
        plsc.subcore_barrier()
        pltpu.sync_copy(acc.at[pl.ds(r0, NPS)], out_hbm.at[cid, pl.ds(r0, NPS)])

        @pl.when(sid == NS - 1)
        def _():
            pltpu.sync_copy(acc.at[pl.ds(NS * NPS, TAIL)],
                            out_hbm.at[cid, pl.ds(NS * NPS, TAIL)])

    @functools.partial(
        pl.kernel,
        out_type=jax.ShapeDtypeStruct((NC, N, 16), f32),
        mesh=mesh,
        scratch_types=[
            pltpu.VMEM((NCHS, CHS), jnp.int32),
            pltpu.VMEM((CHS, 16), f32),
            pltpu.VMEM_SHARED((N, 16), f32),
            pltpu.SemaphoreType.DMA,
        ],
        compiler_params=pltpu.CompilerParams(use_tc_tiling_on_sc=False),
    )
    def sc_counts(dst_hbm, ones_hbm, zeros_hbm, out_hbm, dst_v, buf, acc, sem):
        cid = lax.axis_index("c")
        sid = lax.axis_index("s")
        wid = sid * NC + cid
        r0 = sid * NPS
        pltpu.sync_copy(zeros_hbm.at[pl.ds(r0, NPS)], acc.at[pl.ds(r0, NPS)])

        @pl.when(sid == NS - 1)
        def _():
            pltpu.sync_copy(zeros_hbm.at[pl.ds(NS * NPS, TAIL)],
                            acc.at[pl.ds(NS * NPS, TAIL)])

        plsc.subcore_barrier()

        pltpu.sync_copy(dst_hbm.at[wid], dst_v)
        pltpu.sync_copy(ones_hbm, buf)

        def body(j, carry):
            pltpu.sync_copy(buf, acc.at[dst_v.at[j]], add=True)
            return carry

        lax.fori_loop(0, NCHS, body, 0)
        plsc.subcore_barrier()
        pltpu.sync_copy(acc.at[pl.ds(r0, NPS)], out_hbm.at[cid, pl.ds(r0, NPS)])

        @pl.when(sid == NS - 1)
        def _():
            pltpu.sync_copy(acc.at[pl.ds(NS * NPS, TAIL)],
                            out_hbm.at[cid, pl.ds(NS * NPS, TAIL)])

    return sc_gather, sc_scatter, sc_counts


def _sc_gather(p, q, dst2d, src2d):
    return _sc_kernels()[0](p, q, dst2d, src2d)


def _sc_scatter(m, dst2d, zeros_h):
    return _sc_kernels()[1](m, dst2d, zeros_h)


def _sc_counts(dst2d, ones_ch, zeros_16):
    return _sc_kernels()[2](dst2d, ones_ch, zeros_16)


def _prep_body(x_ref, wa_ref, wb_ref, p_ref, q_ref):
    x = x_ref[...]
    p_ref[...] = jnp.dot(x, wa_ref[...], preferred_element_type=f32)
    q_ref[...] = jnp.dot(x, wb_ref[...], preferred_element_type=f32)


def _tc_prep(x, wa, wb):
    bn = 2000
    cin = x.shape[1]
    return pl.pallas_call(
        _prep_body,
        grid=(N // bn,),
        in_specs=[
            pl.BlockSpec((bn, cin), lambda i: (i, 0)),
            pl.BlockSpec((cin, H), lambda i: (0, 0)),
            pl.BlockSpec((cin, H), lambda i: (0, 0)),
        ],
        out_specs=[
            pl.BlockSpec((bn, H), lambda i: (i, 0)),
            pl.BlockSpec((bn, H), lambda i: (i, 0)),
        ],
        out_shape=[
            jax.ShapeDtypeStruct((N, H), f32),
            jax.ShapeDtypeStruct((N, H), f32),
        ],
    )(x, wa, wb)


def _edge_body(gd_ref, gs_ref, ea_ref, wc_ref, b0_ref, w1_ref, b1_ref,
               wg_ref, bg_ref, m_ref):
    ea = ea_ref[...]
    pre = (gd_ref[...] + gs_ref[...]
           + jnp.dot(ea, wc_ref[...], preferred_element_type=f32)
           + b0_ref[...])
    h = jnp.maximum(pre, 0.0)
    msg = jnp.dot(h, w1_ref[...], preferred_element_type=f32) + b1_ref[...]
    gate = jax.nn.sigmoid(
        jnp.dot(ea, wg_ref[...], preferred_element_type=f32) + bg_ref[...])
    m_ref[...] = msg * gate


def _tc_edge(gd, gs, ea2, wc2, b02, w12, b12, wg2, bg2):
    be = 1000
    e2 = E // 2
    return pl.pallas_call(
        _edge_body,
        grid=(e2 // be,),
        in_specs=[
            pl.BlockSpec((be, 128), lambda i: (i, 0)),
            pl.BlockSpec((be, 128), lambda i: (i, 0)),
            pl.BlockSpec((be, 2 * D_EDGE), lambda i: (i, 0)),
            pl.BlockSpec((2 * D_EDGE, 128), lambda i: (0, 0)),
            pl.BlockSpec((1, 128), lambda i: (0, 0)),
            pl.BlockSpec((128, 128), lambda i: (0, 0)),
            pl.BlockSpec((1, 128), lambda i: (0, 0)),
            pl.BlockSpec((2 * D_EDGE, 128), lambda i: (0, 0)),
            pl.BlockSpec((1, 128), lambda i: (0, 0)),
        ],
        out_specs=pl.BlockSpec((be, 128), lambda i: (i, 0)),
        out_shape=jax.ShapeDtypeStruct((e2, 128), f32),
    )(gd, gs, ea2, wc2, b02, w12, b12, wg2, bg2)


def _make_node_body(has_res, has_next):
    def body(*refs):
        it = iter(refs)
        x_ref = next(it)
        s_ref = next(it)
        cnt_ref = next(it)
        wnx_ref = next(it)
        wna_ref = next(it)
        bn0_ref = next(it)
        wn1_ref = next(it)
        bn1_ref = next(it)
        if has_res:
            rw_ref = next(it)
            rb_ref = next(it)
        g_ref = next(it)
        b_ref = next(it)
        if has_next:
            wa_ref = next(it)
            wb_ref = next(it)
        h_ref = next(it)
        if has_next:
            p_ref = next(it)
            q_ref = next(it)

        x = x_ref[...]
        sums = s_ref[0] + s_ref[1]
        cnt = cnt_ref[0, :, 0:1] + cnt_ref[1, :, 0:1]
        aggr = sums / jnp.maximum(cnt, 1.0)
        u = jnp.maximum(
            jnp.dot(x, wnx_ref[...], preferred_element_type=f32)
            + jnp.dot(aggr, wna_ref[...], preferred_element_type=f32)
            + bn0_ref[...], 0.0)
        out = jnp.dot(u, wn1_ref[...], preferred_element_type=f32) + bn1_ref[...]
        if has_res:
            res = jnp.dot(x, rw_ref[...], preferred_element_type=f32) + rb_ref[...]
        else:
            res = x
        z = out + res
        mu = jnp.mean(z, axis=-1, keepdims=True)
        var = jnp.mean((z - mu) * (z - mu), axis=-1, keepdims=True)
        zn = (z - mu) * lax.rsqrt(var + 1e-5) * g_ref[...] + b_ref[...]
        h = jnp.maximum(zn, 0.0)
        h_ref[...] = h
        if has_next:
            p_ref[...] = jnp.dot(h, wa_ref[...], preferred_element_type=f32)
            q_ref[...] = jnp.dot(h, wb_ref[...], preferred_element_type=f32)
    return body


def _tc_node(x, s01, cnts, p, has_res, nxt):
    bn = 2000
    cin = x.shape[1]
    s = s01
    in_specs = [
        pl.BlockSpec((bn, cin), lambda i: (i, 0)),
        pl.BlockSpec((NC, bn, H), lambda i: (0, i, 0)),
        pl.BlockSpec((NC, bn, 16), lambda i: (0, i, 0)),
        pl.BlockSpec((cin, H), lambda i: (0, 0)),
        pl.BlockSpec((H, H), lambda i: (0, 0)),
        pl.BlockSpec((1, H), lambda i: (0, 0)),
        pl.BlockSpec((H, H), lambda i: (0, 0)),
        pl.BlockSpec((1, H), lambda i: (0, 0)),
    ]
    wnx = p['node_w0'][:cin]
    wna = p['node_w0'][cin:]
    args = [x, s, cnts, wnx, wna, p['node_b0'].reshape(1, H),
            p['node_w1'], p['node_b1'].reshape(1, H)]
    if has_res:
        in_specs += [
            pl.BlockSpec((cin, H), lambda i: (0, 0)),
            pl.BlockSpec((1, H), lambda i: (0, 0)),
        ]
        args += [p['res_w'], p['res_b'].reshape(1, H)]
    in_specs += [
        pl.BlockSpec((1, H), lambda i: (0, 0)),
        pl.BlockSpec((1, H), lambda i: (0, 0)),
    ]
    args += [p['ln_g'].reshape(1, H), p['ln_b'].reshape(1, H)]
    out_specs = [pl.BlockSpec((bn, H), lambda i: (i, 0))]
    out_shape = [jax.ShapeDtypeStruct((N, H), f32)]
    if nxt is not None:
        wa_n, wb_n = nxt
        in_specs += [
            pl.BlockSpec((H, H), lambda i: (0, 0)),
            pl.BlockSpec((H, H), lambda i: (0, 0)),
        ]
        args += [wa_n, wb_n]
        out_specs += [
            pl.BlockSpec((bn, H), lambda i: (i, 0)),
            pl.BlockSpec((bn, H), lambda i: (i, 0)),
        ]
        out_shape += [
            jax.ShapeDtypeStruct((N, H), f32),
            jax.ShapeDtypeStruct((N, H), f32),
        ]
    return pl.pallas_call(
        _make_node_body(has_res, nxt is not None),
        grid=(N // bn,),
        in_specs=in_specs,
        out_specs=out_specs,
        out_shape=out_shape,
    )(*args)


def _readout_body(h_ref, b_ref, w0_ref, b0_ref, w1_ref, b1_ref, o_ref):
    h = h_ref[...]
    ids = b_ref[...]
    onehot = (ids == lax.broadcasted_iota(jnp.int32, (1, G), 1)).astype(f32)
    sums = lax.dot_general(onehot, h, (((0,), (0,)), ((), ())),
                           preferred_element_type=f32,
                           precision=lax.Precision.HIGHEST)
    cnt = jnp.sum(onehot, axis=0, keepdims=True)
    hg = sums / jnp.maximum(cnt.T, 1.0)
    o = jnp.maximum(
        jnp.dot(hg, w0_ref[...], preferred_element_type=f32) + b0_ref[...],
        0.0)
    o_ref[...] = jnp.dot(o, w1_ref[...], preferred_element_type=f32) + b1_ref[...]


def _tc_readout(h, batch2d, r):
    return pl.pallas_call(
        _readout_body,
        grid=(1,),
        in_specs=[
            pl.BlockSpec((N, H), lambda i: (0, 0)),
            pl.BlockSpec((N, 1), lambda i: (0, 0)),
            pl.BlockSpec((H, DEC), lambda i: (0, 0)),
            pl.BlockSpec((1, DEC), lambda i: (0, 0)),
            pl.BlockSpec((DEC, OUT), lambda i: (0, 0)),
            pl.BlockSpec((1, OUT), lambda i: (0, 0)),
        ],
        out_specs=pl.BlockSpec((G, OUT), lambda i: (0, 0)),
        out_shape=jax.ShapeDtypeStruct((G, OUT), f32),
    )(h, batch2d, r['w0'], r['b0'].reshape(1, DEC), r['w1'],
      r['b1'].reshape(1, OUT))


def _blockdiag(w):
    z = jnp.zeros_like(w)
    top = jnp.concatenate([w, z], axis=1)
    bot = jnp.concatenate([z, w], axis=1)
    return jnp.concatenate([top, bot], axis=0)


def _dup(b):
    return jnp.concatenate([b, b]).reshape(1, -1)


def kernel(x, edge_index, edge_attr, batch, params):
    src = edge_index[0]
    dst = edge_index[1]
    dst2d = dst.reshape(NW, NCH, CH)
    src2d = src.reshape(NW, NCH, CH)
    dst2s = dst.reshape(NW, NCHS, CHS)

    ea2 = edge_attr.reshape(E // 2, 2 * D_EDGE)
    zeros_h = jnp.zeros((N, H), f32)
    zeros_16 = jnp.zeros((N, 16), f32)
    ones_ch = jnp.ones((CHS, 16), f32)

    cnts = _sc_counts(dst2s, ones_ch, zeros_16)

    h = x
    p0 = params['layer0']
    cin0 = D_IN
    wa = p0['msg_w0'][:cin0]
    wb = p0['msg_w0'][cin0:2 * cin0]
    P, Q = _tc_prep(x, wa, wb)

    for l in range(NUM_LAYERS):
        p = params['layer%d' % l]
        cin = D_IN if l == 0 else H
        wc = p['msg_w0'][2 * cin:]
        gd, gs = _sc_gather(P, Q, dst2d, src2d)
        m = _tc_edge(gd.reshape(E // 2, 128), gs.reshape(E // 2, 128), ea2,
                     _blockdiag(wc), _dup(p['msg_b0']),
                     _blockdiag(p['msg_w1']), _dup(p['msg_b1']),
                     _blockdiag(p['gate_w']), _dup(p['gate_b']))
        sums = _sc_scatter(m.reshape(E, H), dst2s, zeros_h)
        if l + 1 < NUM_LAYERS:
            pn = params['layer%d' % (l + 1)]
            nxt = (pn['msg_w0'][:H], pn['msg_w0'][H:2 * H])
            h, P, Q = _tc_node(h, sums, cnts, p, l == 0, nxt)
        else:
            (h,) = _tc_node(h, sums, cnts, p, l == 0, None)

    return _tc_readout(h, batch.reshape(N, 1), params['readout'])


# speedup vs baseline: 5.7509x; 1.1100x over previous
import functools

import jax
import jax.numpy as jnp
from jax import lax
from jax.experimental import pallas as pl
from jax.experimental.pallas import tpu as pltpu
from jax.experimental.pallas import tpu_sc as plsc

N = 10000
E = 320000
D_IN = 128
D_EDGE = 16
H = 64
DEC = 64
OUT = 1
G = 256
NUM_LAYERS = 3

NC = 2
NS = 16
NW = NC * NS
EPW = E // NW
CH = 250
NCH = EPW // CH
CHS = 1000
NCHS = EPW // CHS
NPS = 624
TAIL = N - NS * NPS

f32 = jnp.float32


@functools.lru_cache(maxsize=1)
def _sc_kernels():
    mesh = plsc.VectorSubcoreMesh(core_axis_name="c", subcore_axis_name="s")

    @functools.partial(
        pl.kernel,
        out_type=jax.ShapeDtypeStruct((E, H), f32),
        mesh=mesh,
        scratch_types=[
            pltpu.VMEM((NCH, CH), jnp.int32),
            pltpu.VMEM((NCH, CH), jnp.int32),
            pltpu.VMEM((CH, H), f32),
            pltpu.VMEM((CH, H), f32),
            pltpu.VMEM((CH, H), f32),
            pltpu.VMEM((CH, H), f32),
            pltpu.SemaphoreType.DMA,
        ],
        compiler_params=pltpu.CompilerParams(use_tc_tiling_on_sc=False),
    )
    def sc_gather(p_hbm, q_hbm, dst_hbm, src_hbm, out_hbm,
                  dst_v, src_v, bd0, bs0, bd1, bs1, sem):
        wid = lax.axis_index("s") * NC + lax.axis_index("c")
        pltpu.sync_copy(dst_hbm.at[wid], dst_v)
        pltpu.sync_copy(src_hbm.at[wid], src_v)

        def fire(c, bd, bs):
            pltpu.async_copy(p_hbm.at[dst_v.at[c]], bd, sem)
            pltpu.async_copy(q_hbm.at[src_v.at[c]], bs, sem)

        def drain(bd, bs):
            pltpu.make_async_copy(p_hbm.at[pl.ds(0, CH)], bd, sem).wait()
            pltpu.make_async_copy(q_hbm.at[pl.ds(0, CH)], bs, sem).wait()

        def add_write(c, bd, bs):
            def addrow(r, carry):
                for cc in range(H // 16):
                    sl = pl.ds(cc * 16, 16)
                    plsc.addupdate(bd.at[r, sl], bs[r, sl])
                return carry
            lax.fori_loop(0, CH, addrow, 0)
            pltpu.sync_copy(bd, out_hbm.at[pl.ds(wid * EPW + c * CH, CH)])

        fire(0, bd0, bs0)

        def body(i2, carry):
            c0 = 2 * i2
            drain(bd0, bs0)
            fire(c0 + 1, bd1, bs1)
            add_write(c0, bd0, bs0)
            drain(bd1, bs1)

            @pl.when(i2 < NCH // 2 - 1)
            def _():
                fire(c0 + 2, bd0, bs0)

            add_write(c0 + 1, bd1, bs1)
            return carry

        lax.fori_loop(0, NCH // 2, body, 0)

    @functools.partial(
        pl.kernel,
        out_type=jax.ShapeDtypeStruct((NC, N, H), f32),
        mesh=mesh,
        scratch_types=[
            pltpu.VMEM((NCHS, CHS), jnp.int32),
            pltpu.VMEM((CHS, H), f32),
            pltpu.VMEM_SHARED((N, H), f32),
            pltpu.SemaphoreType.DMA,
        ],
        compiler_params=pltpu.CompilerParams(use_tc_tiling_on_sc=False),
    )
    def sc_scatter(m_hbm, dst_hbm, zeros_hbm, out_hbm, dst_v, buf, acc, sem):
        cid = lax.axis_index("c")
        sid = lax.axis_index("s")
        wid = sid * NC + cid
        r0 = sid * NPS
        pltpu.sync_copy(zeros_hbm.at[pl.ds(r0, NPS)], acc.at[pl.ds(r0, NPS)])

        @pl.when(sid == NS - 1)
        def _():
            pltpu.sync_copy(zeros_hbm.at[pl.ds(NS * NPS, TAIL)],
                            acc.at[pl.ds(NS * NPS, TAIL)])

        plsc.subcore_barrier()

        pltpu.sync_copy(dst_hbm.at[wid], dst_v)

        def body(j, carry):
            base = wid * EPW + j * CHS
            pltpu.sync_copy(m_hbm.at[pl.ds(base, CHS)], buf)
            pltpu.sync_copy(buf, acc.at[dst_v.at[j]], add=True)
            return carry

        lax.fori_loop(0, NCHS, body, 0)
        plsc.subcore_barrier()
        pltpu.sync_copy(acc.at[pl.ds(r0, NPS)], out_hbm.at[cid, pl.ds(r0, NPS)])

        @pl.when(sid == NS - 1)
        def _():
            pltpu.sync_copy(acc.at[pl.ds(NS * NPS, TAIL)],
                            out_hbm.at[cid, pl.ds(NS * NPS, TAIL)])

    @functools.partial(
        pl.kernel,
        out_type=jax.ShapeDtypeStruct((NC, N, 16), f32),
        mesh=mesh,
        scratch_types=[
            pltpu.VMEM((NCHS, CHS), jnp.int32),
            pltpu.VMEM((CHS, 16), f32),
            pltpu.VMEM_SHARED((N, 16), f32),
            pltpu.SemaphoreType.DMA,
        ],
        compiler_params=pltpu.CompilerParams(use_tc_tiling_on_sc=False),
    )
    def sc_counts(dst_hbm, ones_hbm, zeros_hbm, out_hbm, dst_v, buf, acc, sem):
        cid = lax.axis_index("c")
        sid = lax.axis_index("s")
        wid = sid * NC + cid
        r0 = sid * NPS
        pltpu.sync_copy(zeros_hbm.at[pl.ds(r0, NPS)], acc.at[pl.ds(r0, NPS)])

        @pl.when(sid == NS - 1)
        def _():
            pltpu.sync_copy(zeros_hbm.at[pl.ds(NS * NPS, TAIL)],
                            acc.at[pl.ds(NS * NPS, TAIL)])

        plsc.subcore_barrier()

        pltpu.sync_copy(dst_hbm.at[wid], dst_v)
        pltpu.sync_copy(ones_hbm, buf)

        def body(j, carry):
            pltpu.sync_copy(buf, acc.at[dst_v.at[j]], add=True)
            return carry

        lax.fori_loop(0, NCHS, body, 0)
        plsc.subcore_barrier()
        pltpu.sync_copy(acc.at[pl.ds(r0, NPS)], out_hbm.at[cid, pl.ds(r0, NPS)])

        @pl.when(sid == NS - 1)
        def _():
            pltpu.sync_copy(acc.at[pl.ds(NS * NPS, TAIL)],
                            out_hbm.at[cid, pl.ds(NS * NPS, TAIL)])

    return sc_gather, sc_scatter, sc_counts


def _sc_gather(p, q, dst2d, src2d):
    return _sc_kernels()[0](p, q, dst2d, src2d)


def _sc_scatter(m, dst2d, zeros_h):
    return _sc_kernels()[1](m, dst2d, zeros_h)


def _sc_counts(dst2d, ones_ch, zeros_16):
    return _sc_kernels()[2](dst2d, ones_ch, zeros_16)


def _prep_body(x_ref, wa_ref, wb_ref, p_ref, q_ref):
    x = x_ref[...]
    p_ref[...] = jnp.dot(x, wa_ref[...], preferred_element_type=f32)
    q_ref[...] = jnp.dot(x, wb_ref[...], preferred_element_type=f32)


def _tc_prep(x, wa, wb):
    bn = 2000
    cin = x.shape[1]
    return pl.pallas_call(
        _prep_body,
        grid=(N // bn,),
        in_specs=[
            pl.BlockSpec((bn, cin), lambda i: (i, 0)),
            pl.BlockSpec((cin, H), lambda i: (0, 0)),
            pl.BlockSpec((cin, H), lambda i: (0, 0)),
        ],
        out_specs=[
            pl.BlockSpec((bn, H), lambda i: (i, 0)),
            pl.BlockSpec((bn, H), lambda i: (i, 0)),
        ],
        out_shape=[
            jax.ShapeDtypeStruct((N, H), f32),
            jax.ShapeDtypeStruct((N, H), f32),
        ],
    )(x, wa, wb)


def _edge_body(g_ref, ea_ref, wc_ref, b0_ref, w1_ref, b1_ref,
               wg_ref, bg_ref, m_ref):
    ea = ea_ref[...]
    pre = (g_ref[...]
           + jnp.dot(ea, wc_ref[...], preferred_element_type=f32)
           + b0_ref[...])
    h = jnp.maximum(pre, 0.0)
    msg = jnp.dot(h, w1_ref[...], preferred_element_type=f32) + b1_ref[...]
    gate = jax.nn.sigmoid(
        jnp.dot(ea, wg_ref[...], preferred_element_type=f32) + bg_ref[...])
    m_ref[...] = msg * gate


def _tc_edge(g, ea2, wc2, b02, w12, b12, wg2, bg2):
    be = 1000
    e2 = E // 2
    return pl.pallas_call(
        _edge_body,
        grid=(e2 // be,),
        in_specs=[
            pl.BlockSpec((be, 128), lambda i: (i, 0)),
            pl.BlockSpec((be, 2 * D_EDGE), lambda i: (i, 0)),
            pl.BlockSpec((2 * D_EDGE, 128), lambda i: (0, 0)),
            pl.BlockSpec((1, 128), lambda i: (0, 0)),
            pl.BlockSpec((128, 128), lambda i: (0, 0)),
            pl.BlockSpec((1, 128), lambda i: (0, 0)),
            pl.BlockSpec((2 * D_EDGE, 128), lambda i: (0, 0)),
            pl.BlockSpec((1, 128), lambda i: (0, 0)),
        ],
        out_specs=pl.BlockSpec((be, 128), lambda i: (i, 0)),
        out_shape=jax.ShapeDtypeStruct((e2, 128), f32),
    )(g, ea2, wc2, b02, w12, b12, wg2, bg2)


def _make_node_body(has_res, has_next):
    def body(*refs):
        it = iter(refs)
        x_ref = next(it)
        s_ref = next(it)
        cnt_ref = next(it)
        wnx_ref = next(it)
        wna_ref = next(it)
        bn0_ref = next(it)
        wn1_ref = next(it)
        bn1_ref = next(it)
        if has_res:
            rw_ref = next(it)
            rb_ref = next(it)
        g_ref = next(it)
        b_ref = next(it)
        if has_next:
            wa_ref = next(it)
            wb_ref = next(it)
        h_ref = next(it)
        if has_next:
            p_ref = next(it)
            q_ref = next(it)

        x = x_ref[...]
        sums = s_ref[0] + s_ref[1]
        cnt = cnt_ref[0, :, 0:1] + cnt_ref[1, :, 0:1]
        aggr = sums / jnp.maximum(cnt, 1.0)
        u = jnp.maximum(
            jnp.dot(x, wnx_ref[...], preferred_element_type=f32)
            + jnp.dot(aggr, wna_ref[...], preferred_element_type=f32)
            + bn0_ref[...], 0.0)
        out = jnp.dot(u, wn1_ref[...], preferred_element_type=f32) + bn1_ref[...]
        if has_res:
            res = jnp.dot(x, rw_ref[...], preferred_element_type=f32) + rb_ref[...]
        else:
            res = x
        z = out + res
        mu = jnp.mean(z, axis=-1, keepdims=True)
        var = jnp.mean((z - mu) * (z - mu), axis=-1, keepdims=True)
        zn = (z - mu) * lax.rsqrt(var + 1e-5) * g_ref[...] + b_ref[...]
        h = jnp.maximum(zn, 0.0)
        h_ref[...] = h
        if has_next:
            p_ref[...] = jnp.dot(h, wa_ref[...], preferred_element_type=f32)
            q_ref[...] = jnp.dot(h, wb_ref[...], preferred_element_type=f32)
    return body


def _tc_node(x, s01, cnts, p, has_res, nxt):
    bn = 2000
    cin = x.shape[1]
    s = s01
    in_specs = [
        pl.BlockSpec((bn, cin), lambda i: (i, 0)),
        pl.BlockSpec((NC, bn, H), lambda i: (0, i, 0)),
        pl.BlockSpec((NC, bn, 16), lambda i: (0, i, 0)),
        pl.BlockSpec((cin, H), lambda i: (0, 0)),
        pl.BlockSpec((H, H), lambda i: (0, 0)),
        pl.BlockSpec((1, H), lambda i: (0, 0)),
        pl.BlockSpec((H, H), lambda i: (0, 0)),
        pl.BlockSpec((1, H), lambda i: (0, 0)),
    ]
    wnx = p['node_w0'][:cin]
    wna = p['node_w0'][cin:]
    args = [x, s, cnts, wnx, wna, p['node_b0'].reshape(1, H),
            p['node_w1'], p['node_b1'].reshape(1, H)]
    if has_res:
        in_specs += [
            pl.BlockSpec((cin, H), lambda i: (0, 0)),
            pl.BlockSpec((1, H), lambda i: (0, 0)),
        ]
        args += [p['res_w'], p['res_b'].reshape(1, H)]
    in_specs += [
        pl.BlockSpec((1, H), lambda i: (0, 0)),
        pl.BlockSpec((1, H), lambda i: (0, 0)),
    ]
    args += [p['ln_g'].reshape(1, H), p['ln_b'].reshape(1, H)]
    out_specs = [pl.BlockSpec((bn, H), lambda i: (i, 0))]
    out_shape = [jax.ShapeDtypeStruct((N, H), f32)]
    if nxt is not None:
        wa_n, wb_n = nxt
        in_specs += [
            pl.BlockSpec((H, H), lambda i: (0, 0)),
            pl.BlockSpec((H, H), lambda i: (0, 0)),
        ]
        args += [wa_n, wb_n]
        out_specs += [
            pl.BlockSpec((bn, H), lambda i: (i, 0)),
            pl.BlockSpec((bn, H), lambda i: (i, 0)),
        ]
        out_shape += [
            jax.ShapeDtypeStruct((N, H), f32),
            jax.ShapeDtypeStruct((N, H), f32),
        ]
    return pl.pallas_call(
        _make_node_body(has_res, nxt is not None),
        grid=(N // bn,),
        in_specs=in_specs,
        out_specs=out_specs,
        out_shape=out_shape,
    )(*args)


def _readout_body(h_ref, b_ref, w0_ref, b0_ref, w1_ref, b1_ref, o_ref):
    h = h_ref[...]
    ids = b_ref[...]
    onehot = (ids == lax.broadcasted_iota(jnp.int32, (1, G), 1)).astype(f32)
    sums = lax.dot_general(onehot, h, (((0,), (0,)), ((), ())),
                           preferred_element_type=f32,
                           precision=lax.Precision.HIGHEST)
    cnt = jnp.sum(onehot, axis=0, keepdims=True)
    hg = sums / jnp.maximum(cnt.T, 1.0)
    o = jnp.maximum(
        jnp.dot(hg, w0_ref[...], preferred_element_type=f32) + b0_ref[...],
        0.0)
    o_ref[...] = jnp.dot(o, w1_ref[...], preferred_element_type=f32) + b1_ref[...]


def _tc_readout(h, batch2d, r):
    return pl.pallas_call(
        _readout_body,
        grid=(1,),
        in_specs=[
            pl.BlockSpec((N, H), lambda i: (0, 0)),
            pl.BlockSpec((N, 1), lambda i: (0, 0)),
            pl.BlockSpec((H, DEC), lambda i: (0, 0)),
            pl.BlockSpec((1, DEC), lambda i: (0, 0)),
            pl.BlockSpec((DEC, OUT), lambda i: (0, 0)),
            pl.BlockSpec((1, OUT), lambda i: (0, 0)),
        ],
        out_specs=pl.BlockSpec((G, OUT), lambda i: (0, 0)),
        out_shape=jax.ShapeDtypeStruct((G, OUT), f32),
    )(h, batch2d, r['w0'], r['b0'].reshape(1, DEC), r['w1'],
      r['b1'].reshape(1, OUT))


def _blockdiag(w):
    z = jnp.zeros_like(w)
    top = jnp.concatenate([w, z], axis=1)
    bot = jnp.concatenate([z, w], axis=1)
    return jnp.concatenate([top, bot], axis=0)


def _dup(b):
    return jnp.concatenate([b, b]).reshape(1, -1)


def kernel(x, edge_index, edge_attr, batch, params):
    src = edge_index[0]
    dst = edge_index[1]
    dst2d = dst.reshape(NW, NCH, CH)
    src2d = src.reshape(NW, NCH, CH)
    dst2s = dst.reshape(NW, NCHS, CHS)

    ea2 = edge_attr.reshape(E // 2, 2 * D_EDGE)
    zeros_h = jnp.zeros((N, H), f32)
    zeros_16 = jnp.zeros((N, 16), f32)
    ones_ch = jnp.ones((CHS, 16), f32)

    cnts = _sc_counts(dst2s, ones_ch, zeros_16)

    h = x
    p0 = params['layer0']
    cin0 = D_IN
    wa = p0['msg_w0'][:cin0]
    wb = p0['msg_w0'][cin0:2 * cin0]
    P, Q = _tc_prep(x, wa, wb)

    for l in range(NUM_LAYERS):
        p = params['layer%d' % l]
        cin = D_IN if l == 0 else H
        wc = p['msg_w0'][2 * cin:]
        g = _sc_gather(P, Q, dst2d, src2d)
        m = _tc_edge(g.reshape(E // 2, 128), ea2,
                     _blockdiag(wc), _dup(p['msg_b0']),
                     _blockdiag(p['msg_w1']), _dup(p['msg_b1']),
                     _blockdiag(p['gate_w']), _dup(p['gate_b']))
        sums = _sc_scatter(m.reshape(E, H), dst2s, zeros_h)
        if l + 1 < NUM_LAYERS:
            pn = params['layer%d' % (l + 1)]
            nxt = (pn['msg_w0'][:H], pn['msg_w0'][H:2 * H])
            h, P, Q = _tc_node(h, sums, cnts, p, l == 0, nxt)
        else:
            (h,) = _tc_node(h, sums, cnts, p, l == 0, None)

    return _tc_readout(h, batch.reshape(N, 1), params['readout'])


# speedup vs baseline: 6.2463x; 1.0861x over previous
import functools

import jax
import jax.numpy as jnp
from jax import lax
from jax.experimental import pallas as pl
from jax.experimental.pallas import tpu as pltpu
from jax.experimental.pallas import tpu_sc as plsc

N = 10000
E = 320000
D_IN = 128
D_EDGE = 16
H = 64
DEC = 64
OUT = 1
G = 256
NUM_LAYERS = 3

NC = 2
NS = 16
NW = NC * NS
EPW = E // NW
CH = 250
NCH = EPW // CH
CHS = 500
NCHS = EPW // CHS
EB = 1600
NPS = 624
TAIL = N - NS * NPS

f32 = jnp.float32


@functools.lru_cache(maxsize=1)
def _sc_kernels():
    mesh = plsc.VectorSubcoreMesh(core_axis_name="c", subcore_axis_name="s")

    @functools.partial(
        pl.kernel,
        out_type=jax.ShapeDtypeStruct((E, H), f32),
        mesh=mesh,
        scratch_types=[
            pltpu.VMEM((NCH, CH), jnp.int32),
            pltpu.VMEM((NCH, CH), jnp.int32),
            pltpu.VMEM((CH, H), f32),
            pltpu.VMEM((CH, H), f32),
            pltpu.VMEM((CH, H), f32),
            pltpu.VMEM((CH, H), f32),
            pltpu.SemaphoreType.DMA,
        ],
        compiler_params=pltpu.CompilerParams(use_tc_tiling_on_sc=False),
    )
    def sc_gather(p_hbm, q_hbm, dst_hbm, src_hbm, out_hbm,
                  dst_v, src_v, bd0, bs0, bd1, bs1, sem):
        wid = lax.axis_index("s") * NC + lax.axis_index("c")
        pltpu.sync_copy(dst_hbm.at[wid], dst_v)
        pltpu.sync_copy(src_hbm.at[wid], src_v)

        def fire(c, bd, bs):
            pltpu.async_copy(p_hbm.at[dst_v.at[c]], bd, sem)
            pltpu.async_copy(q_hbm.at[src_v.at[c]], bs, sem)

        def drain(bd, bs):
            pltpu.make_async_copy(p_hbm.at[pl.ds(0, CH)], bd, sem).wait()
            pltpu.make_async_copy(q_hbm.at[pl.ds(0, CH)], bs, sem).wait()

        def add_write(c, bd, bs):
            def addrow(r, carry):
                for cc in range(H // 16):
                    sl = pl.ds(cc * 16, 16)
                    plsc.addupdate(bd.at[r, sl], bs[r, sl])
                return carry
            lax.fori_loop(0, CH, addrow, 0)
            pltpu.sync_copy(bd, out_hbm.at[pl.ds(wid * EPW + c * CH, CH)])

        fire(0, bd0, bs0)

        def body(i2, carry):
            c0 = 2 * i2
            drain(bd0, bs0)
            fire(c0 + 1, bd1, bs1)
            add_write(c0, bd0, bs0)
            drain(bd1, bs1)

            @pl.when(i2 < NCH // 2 - 1)
            def _():
                fire(c0 + 2, bd0, bs0)

            add_write(c0 + 1, bd1, bs1)
            return carry

        lax.fori_loop(0, NCH // 2, body, 0)

    @functools.partial(
        pl.kernel,
        out_type=jax.ShapeDtypeStruct((NC, N, H), f32),
        mesh=mesh,
        scratch_types=[
            pltpu.VMEM((NCHS, CHS), jnp.int32),
            pltpu.VMEM((CHS, H), f32),
            pltpu.VMEM((CHS, H), f32),
            pltpu.VMEM_SHARED((N, H), f32),
            pltpu.SemaphoreType.DMA,
        ],
        compiler_params=pltpu.CompilerParams(use_tc_tiling_on_sc=False),
    )
    def sc_scatter(m_hbm, dst_hbm, zeros_hbm, out_hbm, dst_v, buf, buf1,
                   acc, sem):
        cid = lax.axis_index("c")
        sid = lax.axis_index("s")
        wid = sid * NC + cid
        r0 = sid * NPS
        pltpu.sync_copy(zeros_hbm.at[pl.ds(r0, NPS)], acc.at[pl.ds(r0, NPS)])

        @pl.when(sid == NS - 1)
        def _():
            pltpu.sync_copy(zeros_hbm.at[pl.ds(NS * NPS, TAIL)],
                            acc.at[pl.ds(NS * NPS, TAIL)])

        plsc.subcore_barrier()

        pltpu.sync_copy(dst_hbm.at[wid], dst_v)

        def fire(c, b):
            pltpu.async_copy(m_hbm.at[pl.ds(wid * EPW + c * CHS, CHS)], b, sem)

        def drain(b):
            pltpu.make_async_copy(m_hbm.at[pl.ds(0, CHS)], b, sem).wait()

        fire(0, buf)

        def body(i2, carry):
            c0 = 2 * i2
            drain(buf)
            fire(c0 + 1, buf1)
            pltpu.sync_copy(buf, acc.at[dst_v.at[c0]], add=True)
            drain(buf1)

            @pl.when(i2 < NCHS // 2 - 1)
            def _():
                fire(c0 + 2, buf)

            pltpu.sync_copy(buf1, acc.at[dst_v.at[c0 + 1]], add=True)
            return carry

        lax.fori_loop(0, NCHS // 2, body, 0)
        plsc.subcore_barrier()
        pltpu.sync_copy(acc.at[pl.ds(r0, NPS)], out_hbm.at[cid, pl.ds(r0, NPS)])

        @pl.when(sid == NS - 1)
        def _():
            pltpu.sync_copy(acc.at[pl.ds(NS * NPS, TAIL)],
                            out_hbm.at[cid, pl.ds(NS * NPS, TAIL)])

    @functools.partial(
        pl.kernel,
        out_type=jax.ShapeDtypeStruct((NC, N, 16), f32),
        mesh=mesh,
        scratch_types=[
            pltpu.VMEM((NCHS, CHS), jnp.int32),
            pltpu.VMEM((CHS, 16), f32),
            pltpu.VMEM_SHARED((N, 16), f32),
            pltpu.SemaphoreType.DMA,
        ],
        compiler_params=pltpu.CompilerParams(use_tc_tiling_on_sc=False),
    )
    def sc_counts(dst_hbm, ones_hbm, zeros_hbm, out_hbm, dst_v, buf, acc, sem):
        cid = lax.axis_index("c")
        sid = lax.axis_index("s")
        wid = sid * NC + cid
        r0 = sid * NPS
        pltpu.sync_copy(zeros_hbm.at[pl.ds(r0, NPS)], acc.at[pl.ds(r0, NPS)])

        @pl.when(sid == NS - 1)
        def _():
            pltpu.sync_copy(zeros_hbm.at[pl.ds(NS * NPS, TAIL)],
                            acc.at[pl.ds(NS * NPS, TAIL)])

        plsc.subcore_barrier()

        pltpu.sync_copy(dst_hbm.at[wid], dst_v)
        pltpu.sync_copy(ones_hbm, buf)

        def body(j, carry):
            pltpu.sync_copy(buf, acc.at[dst_v.at[j]], add=True)
            return carry

        lax.fori_loop(0, NCHS, body, 0)
        plsc.subcore_barrier()
        pltpu.sync_copy(acc.at[pl.ds(r0, NPS)], out_hbm.at[cid, pl.ds(r0, NPS)])

        @pl.when(sid == NS - 1)
        def _():
            pltpu.sync_copy(acc.at[pl.ds(NS * NPS, TAIL)],
                            out_hbm.at[cid, pl.ds(NS * NPS, TAIL)])

    return sc_gather, sc_scatter, sc_counts


def _sc_gather(p, q, dst2d, src2d):
    return _sc_kernels()[0](p, q, dst2d, src2d)


def _sc_scatter(m, dst2d, zeros_h):
    return _sc_kernels()[1](m, dst2d, zeros_h)


def _sc_counts(dst2d, ones_ch, zeros_16):
    return _sc_kernels()[2](dst2d, ones_ch, zeros_16)


def _prep_body(x_ref, wa_ref, wb_ref, p_ref, q_ref):
    x = x_ref[...]
    p_ref[...] = jnp.dot(x, wa_ref[...], preferred_element_type=f32)
    q_ref[...] = jnp.dot(x, wb_ref[...], preferred_element_type=f32)


def _tc_prep(x, wa, wb):
    bn = 2000
    cin = x.shape[1]
    return pl.pallas_call(
        _prep_body,
        grid=(N // bn,),
        in_specs=[
            pl.BlockSpec((bn, cin), lambda i: (i, 0)),
            pl.BlockSpec((cin, H), lambda i: (0, 0)),
            pl.BlockSpec((cin, H), lambda i: (0, 0)),
        ],
        out_specs=[
            pl.BlockSpec((bn, H), lambda i: (i, 0)),
            pl.BlockSpec((bn, H), lambda i: (i, 0)),
        ],
        out_shape=[
            jax.ShapeDtypeStruct((N, H), f32),
            jax.ShapeDtypeStruct((N, H), f32),
        ],
    )(x, wa, wb)


def _edge_body(g_ref, ea_ref, wc_ref, b0_ref, w1_ref, b1_ref,
               wg_ref, bg_ref, m_ref):
    t = ea_ref[...]
    q = t.shape[0]
    ea = jnp.concatenate([t[:, 32 * k:32 * (k + 1)] for k in range(4)],
                         axis=0)
    pre = (g_ref[...]
           + jnp.dot(ea, wc_ref[...], preferred_element_type=f32)
           + b0_ref[...])
    h = jnp.maximum(pre, 0.0)
    msg = jnp.dot(h, w1_ref[...], preferred_element_type=f32) + b1_ref[...]
    gate = jax.nn.sigmoid(
        jnp.dot(ea, wg_ref[...], preferred_element_type=f32) + bg_ref[...])
    m_ref[...] = msg * gate


def _tc_edge(g, ea8, wc2, b02, w12, b12, wg2, bg2):
    be = EB
    e2 = E // 2
    return pl.pallas_call(
        _edge_body,
        grid=(e2 // be,),
        in_specs=[
            pl.BlockSpec((be, 128), lambda i: (i, 0)),
            pl.BlockSpec((be // 4, 128), lambda i: (i, 0)),
            pl.BlockSpec((2 * D_EDGE, 128), lambda i: (0, 0)),
            pl.BlockSpec((1, 128), lambda i: (0, 0)),
            pl.BlockSpec((128, 128), lambda i: (0, 0)),
            pl.BlockSpec((1, 128), lambda i: (0, 0)),
            pl.BlockSpec((2 * D_EDGE, 128), lambda i: (0, 0)),
            pl.BlockSpec((1, 128), lambda i: (0, 0)),
        ],
        out_specs=pl.BlockSpec((be, 128), lambda i: (i, 0)),
        out_shape=jax.ShapeDtypeStruct((e2, 128), f32),
    )(g, ea8, wc2, b02, w12, b12, wg2, bg2)


def _make_node_body(has_res, has_next):
    def body(*refs):
        it = iter(refs)
        x_ref = next(it)
        s_ref = next(it)
        cnt_ref = next(it)
        wnx_ref = next(it)
        wna_ref = next(it)
        bn0_ref = next(it)
        wn1_ref = next(it)
        bn1_ref = next(it)
        if has_res:
            rw_ref = next(it)
            rb_ref = next(it)
        g_ref = next(it)
        b_ref = next(it)
        if has_next:
            wa_ref = next(it)
            wb_ref = next(it)
        h_ref = next(it)
        if has_next:
            p_ref = next(it)
            q_ref = next(it)

        x = x_ref[...]
        sums = s_ref[0] + s_ref[1]
        cnt = cnt_ref[0, :, 0:1] + cnt_ref[1, :, 0:1]
        aggr = sums / jnp.maximum(cnt, 1.0)
        u = jnp.maximum(
            jnp.dot(x, wnx_ref[...], preferred_element_type=f32)
            + jnp.dot(aggr, wna_ref[...], preferred_element_type=f32)
            + bn0_ref[...], 0.0)
        out = jnp.dot(u, wn1_ref[...], preferred_element_type=f32) + bn1_ref[...]
        if has_res:
            res = jnp.dot(x, rw_ref[...], preferred_element_type=f32) + rb_ref[...]
        else:
            res = x
        z = out + res
        mu = jnp.mean(z, axis=-1, keepdims=True)
        var = jnp.mean((z - mu) * (z - mu), axis=-1, keepdims=True)
        zn = (z - mu) * lax.rsqrt(var + 1e-5) * g_ref[...] + b_ref[...]
        h = jnp.maximum(zn, 0.0)
        h_ref[...] = h
        if has_next:
            p_ref[...] = jnp.dot(h, wa_ref[...], preferred_element_type=f32)
            q_ref[...] = jnp.dot(h, wb_ref[...], preferred_element_type=f32)
    return body


def _tc_node(x, s01, cnts, p, has_res, nxt):
    bn = 2000
    cin = x.shape[1]
    s = s01
    in_specs = [
        pl.BlockSpec((bn, cin), lambda i: (i, 0)),
        pl.BlockSpec((NC, bn, H), lambda i: (0, i, 0)),
        pl.BlockSpec((NC, bn, 16), lambda i: (0, i, 0)),
        pl.BlockSpec((cin, H), lambda i: (0, 0)),
        pl.BlockSpec((H, H), lambda i: (0, 0)),
        pl.BlockSpec((1, H), lambda i: (0, 0)),
        pl.BlockSpec((H, H), lambda i: (0, 0)),
        pl.BlockSpec((1, H), lambda i: (0, 0)),
    ]
    wnx = p['node_w0'][:cin]
    wna = p['node_w0'][cin:]
    args = [x, s, cnts, wnx, wna, p['node_b0'].reshape(1, H),
            p['node_w1'], p['node_b1'].reshape(1, H)]
    if has_res:
        in_specs += [
            pl.BlockSpec((cin, H), lambda i: (0, 0)),
            pl.BlockSpec((1, H), lambda i: (0, 0)),
        ]
        args += [p['res_w'], p['res_b'].reshape(1, H)]
    in_specs += [
        pl.BlockSpec((1, H), lambda i: (0, 0)),
        pl.BlockSpec((1, H), lambda i: (0, 0)),
    ]
    args += [p['ln_g'].reshape(1, H), p['ln_b'].reshape(1, H)]
    out_specs = [pl.BlockSpec((bn, H), lambda i: (i, 0))]
    out_shape = [jax.ShapeDtypeStruct((N, H), f32)]
    if nxt is not None:
        wa_n, wb_n = nxt
        in_specs += [
            pl.BlockSpec((H, H), lambda i: (0, 0)),
            pl.BlockSpec((H, H), lambda i: (0, 0)),
        ]
        args += [wa_n, wb_n]
        out_specs += [
            pl.BlockSpec((bn, H), lambda i: (i, 0)),
            pl.BlockSpec((bn, H), lambda i: (i, 0)),
        ]
        out_shape += [
            jax.ShapeDtypeStruct((N, H), f32),
            jax.ShapeDtypeStruct((N, H), f32),
        ]
    return pl.pallas_call(
        _make_node_body(has_res, nxt is not None),
        grid=(N // bn,),
        in_specs=in_specs,
        out_specs=out_specs,
        out_shape=out_shape,
    )(*args)


def _readout_body(h_ref, b_ref, w0_ref, b0_ref, w1_ref, b1_ref, o_ref):
    h = h_ref[...]
    ids = b_ref[...]
    onehot = (ids == lax.broadcasted_iota(jnp.int32, (1, G), 1)).astype(f32)
    sums = lax.dot_general(onehot, h, (((0,), (0,)), ((), ())),
                           preferred_element_type=f32,
                           precision=lax.Precision.HIGHEST)
    cnt = jnp.sum(onehot, axis=0, keepdims=True)
    hg = sums / jnp.maximum(cnt.T, 1.0)
    o = jnp.maximum(
        jnp.dot(hg, w0_ref[...], preferred_element_type=f32) + b0_ref[...],
        0.0)
    o_ref[...] = jnp.dot(o, w1_ref[...], preferred_element_type=f32) + b1_ref[...]


def _tc_readout(h, batch2d, r):
    return pl.pallas_call(
        _readout_body,
        grid=(1,),
        in_specs=[
            pl.BlockSpec((N, H), lambda i: (0, 0)),
            pl.BlockSpec((N, 1), lambda i: (0, 0)),
            pl.BlockSpec((H, DEC), lambda i: (0, 0)),
            pl.BlockSpec((1, DEC), lambda i: (0, 0)),
            pl.BlockSpec((DEC, OUT), lambda i: (0, 0)),
            pl.BlockSpec((1, OUT), lambda i: (0, 0)),
        ],
        out_specs=pl.BlockSpec((G, OUT), lambda i: (0, 0)),
        out_shape=jax.ShapeDtypeStruct((G, OUT), f32),
    )(h, batch2d, r['w0'], r['b0'].reshape(1, DEC), r['w1'],
      r['b1'].reshape(1, OUT))


def _blockdiag(w):
    z = jnp.zeros_like(w)
    top = jnp.concatenate([w, z], axis=1)
    bot = jnp.concatenate([z, w], axis=1)
    return jnp.concatenate([top, bot], axis=0)


def _dup(b):
    return jnp.concatenate([b, b]).reshape(1, -1)


def kernel(x, edge_index, edge_attr, batch, params):
    src = edge_index[0]
    dst = edge_index[1]
    dst2d = dst.reshape(NW, NCH, CH)
    src2d = src.reshape(NW, NCH, CH)
    dst2s = dst.reshape(NW, NCHS, CHS)

    nb = (E // 2) // EB
    ea8 = (edge_attr.reshape(nb, 4, EB // 4, 32)
           .transpose(0, 2, 1, 3).reshape(E // 8, 128))
    zeros_h = jnp.zeros((N, H), f32)
    zeros_16 = jnp.zeros((N, 16), f32)
    ones_ch = jnp.ones((CHS, 16), f32)

    cnts = _sc_counts(dst2s, ones_ch, zeros_16)

    h = x
    p0 = params['layer0']
    cin0 = D_IN
    wa = p0['msg_w0'][:cin0]
    wb = p0['msg_w0'][cin0:2 * cin0]
    P, Q = _tc_prep(x, wa, wb)

    for l in range(NUM_LAYERS):
        p = params['layer%d' % l]
        cin = D_IN if l == 0 else H
        wc = p['msg_w0'][2 * cin:]
        g = _sc_gather(P, Q, dst2d, src2d)
        m = _tc_edge(g.reshape(E // 2, 128), ea8,
                     _blockdiag(wc), _dup(p['msg_b0']),
                     _blockdiag(p['msg_w1']), _dup(p['msg_b1']),
                     _blockdiag(p['gate_w']), _dup(p['gate_b']))
        sums = _sc_scatter(m.reshape(E, H), dst2s, zeros_h)
        if l + 1 < NUM_LAYERS:
            pn = params['layer%d' % (l + 1)]
            nxt = (pn['msg_w0'][:H], pn['msg_w0'][H:2 * H])
            h, P, Q = _tc_node(h, sums, cnts, p, l == 0, nxt)
        else:
            (h,) = _tc_node(h, sums, cnts, p, l == 0, None)

    return _tc_readout(h, batch.reshape(N, 1), params['readout'])
